# baseline (speedup 1.0000x reference)
import jax
import jax.numpy as jnp
from jax import lax
from jax.experimental import pallas as pl
from jax.experimental.pallas import tpu as pltpu
from jax.experimental.pallas import tpu_sc as plsc

_B = 64
_NC = 2
_NS = 16
_LANES = 16


def _lane_gather(vec, idx):
    return lax.gather(
        vec,
        idx[:, None],
        dimension_numbers=lax.GatherDimensionNumbers(
            offset_dims=(), collapsed_slice_dims=(0,), start_index_map=(0,)),
        slice_sizes=(1,),
        mode=lax.GatherScatterMode.PROMISE_IN_BOUNDS,
    )


def _proj_body(h_ref, wz_ref, a_ref, z_ref, st_ref):
    hb = h_ref[...]
    z = jnp.dot(hb, wz_ref[...], preferred_element_type=jnp.float32)
    z_ref[...] = z
    st_ref[...] = jnp.dot(z, a_ref[...], preferred_element_type=jnp.float32)


def _comb_body(h_ref, a_ref, d_ref, p_ref, o_ref):
    num = a_ref[...]
    d = d_ref[...]
    r = jnp.where(d > 0, 1.0 / jnp.where(d > 0, d, 1.0), 0.0)
    rexp = jnp.dot(r, p_ref[...], preferred_element_type=jnp.float32)
    hag = num * rexp
    o_ref[...] = h_ref[...] + jnp.where(hag > 0, hag, jnp.exp(hag) - 1.0)


def kernel(h, edge_index, e, W_fc, W_attn):
    N, D = h.shape
    E = edge_index.shape[1]
    H, O, _ = W_fc.shape
    HO = H * O
    SW = 2 * H
    STW = HO
    HALF = N // _NC
    DUMA = ((HALF + 127) // 128) * 128
    AROWS = DUMA + 128
    ZA = AROWS // _NS
    DUMD = DUMA // 8
    DROWS = DUMD + 128
    ZD = DROWS // _NS
    NCHUNK = E // _B
    NK = (NCHUNK + _NS - 1) // _NS
    NPAIR = (NK + 1) // 2
    GE = _B // _LANES

    Wz = W_fc.reshape(HO, D).T
    r_idx = jnp.arange(HO)
    hh = r_idx // O
    oo = r_idx % O
    j16 = jnp.arange(SW)
    w1 = W_attn[hh, oo]
    w2 = W_attn[hh, O + oo]
    A = (jnp.where(j16[None, :] == hh[:, None], w1[:, None], 0.0)
         + jnp.where(j16[None, :] == (H + hh)[:, None], w2[:, None], 0.0))
    A = jnp.pad(A, ((0, 0), (0, STW - SW)))

    bn = 1000
    z_flat, st = pl.pallas_call(
        _proj_body,
        grid=(N // bn,),
        in_specs=[
            pl.BlockSpec((bn, D), lambda i: (i, 0)),
            pl.BlockSpec((D, HO), lambda i: (0, 0)),
            pl.BlockSpec((HO, STW), lambda i: (0, 0)),
        ],
        out_specs=[
            pl.BlockSpec((bn, HO), lambda i: (i, 0)),
            pl.BlockSpec((bn, STW), lambda i: (i, 0)),
        ],
        out_shape=[
            jax.ShapeDtypeStruct((N, HO), jnp.float32),
            jax.ShapeDtypeStruct((N, STW), jnp.float32),
        ],
    )(h, Wz, A)

    src = edge_index[0]
    dst = edge_index[1]

    def sc_body(src_h, dst_h, st_h, z_h, acc_o, den_o, *refs):
        (src_v0, dst_v0, dstm_v0, dstd8_v0, sts_v0, std_v0, zrows_v0,
         wout_v0, den_v0,
         src_v1, dst_v1, dstm_v1, dstd8_v1, sts_v1, std_v1, zrows_v1,
         wout_v1, den_v1,
         acc_s, den_s, semi0, semi1, semg0, semg1) = refs
        bufs = (
            (src_v0, dst_v0, dstm_v0, dstd8_v0, sts_v0, std_v0, zrows_v0,
             wout_v0, den_v0, semi0, semg0),
            (src_v1, dst_v1, dstm_v1, dstd8_v1, sts_v1, std_v1, zrows_v1,
             wout_v1, den_v1, semi1, semg1),
        )
        c = lax.axis_index("c")
        s = lax.axis_index("s")

        def zrow_body(r, zcarry):
            for k2 in range(HO // _LANES):
                wout_v0[r, pl.ds(k2 * _LANES, _LANES)] = jnp.zeros(
                    (_LANES,), jnp.float32)
            return zcarry

        lax.fori_loop(0, _B, zrow_body, 0)
        for blk in range(ZA // _B):
            pltpu.sync_copy(wout_v0, acc_s.at[pl.ds(s * ZA + blk * _B, _B)])
        rem = ZA % _B
        if rem:
            pltpu.sync_copy(
                wout_v0.at[pl.ds(0, rem)],
                acc_s.at[pl.ds(s * ZA + (ZA // _B) * _B, rem)])
        for blk in range(ZD // _B):
            pltpu.sync_copy(wout_v0, den_s.at[pl.ds(s * ZD + blk * _B, _B)])
        remd = ZD % _B
        if remd:
            pltpu.sync_copy(
                wout_v0.at[pl.ds(0, remd)],
                den_s.at[pl.ds(s * ZD + (ZD // _B) * _B, remd)])
        plsc.subcore_barrier()

        def issue_idx(t, bf):
            base = (s + t * _NS) * _B
            pltpu.async_copy(src_h.at[pl.ds(base, _B)], bf[0], bf[9])
            pltpu.async_copy(dst_h.at[pl.ds(base, _B)], bf[1], bf[9])

        def wait_idx(bf):
            pltpu.make_async_copy(src_h.at[pl.ds(0, _B)], bf[0], bf[9]).wait()
            pltpu.make_async_copy(dst_h.at[pl.ds(0, _B)], bf[1], bf[9]).wait()

        def issue_gathers(bf):
            pltpu.async_copy(st_h.at[bf[0]], bf[4], bf[10])
            pltpu.async_copy(st_h.at[bf[1]], bf[5], bf[10])
            pltpu.async_copy(z_h.at[bf[0]], bf[6], bf[10])

        def wait_gathers(bf):
            pltpu.make_async_copy(st_h.at[bf[0]], bf[4], bf[10]).wait()
            pltpu.make_async_copy(st_h.at[bf[1]], bf[5], bf[10]).wait()
            pltpu.make_async_copy(z_h.at[bf[0]], bf[6], bf[10]).wait()

        def compute(bf):
            def grp_body(g, gcarry):
                dv = bf[1][pl.ds(g * _LANES, _LANES)]
                my = dv - c * HALF
                val = jnp.clip(my + 1, 0, 1) * jnp.clip(HALF - my, 0, 1)
                inv = 1 - val
                bf[2][pl.ds(g * _LANES, _LANES)] = val * my + inv * DUMA
                bf[3][pl.ds(g * _LANES, _LANES)] = (
                    val * lax.shift_right_logical(my, 3) + inv * DUMD)
                return gcarry

            lax.fori_loop(0, GE, grp_body, 0, unroll=GE)

            def edge_body(ei, ecarry):
                lane = lax.broadcasted_iota(jnp.int32, (_LANES,), 0)
                rot = (lane + H) % _LANES
                lanef = lane.astype(jnp.float32)
                maskf = jnp.clip(float(H) - lanef, 0.0, 1.0)
                bcast = [jnp.full((_LANES,), k2, jnp.int32) for k2 in range(H)]
                a = bf[4][ei, pl.ds(0, _LANES)]
                b = bf[5][ei, pl.ds(0, _LANES)]
                t = a + _lane_gather(b, rot)
                lr = jnp.minimum(jnp.maximum(t, 0.01 * t), 50.0)
                ex = jnp.exp(lr) * maskf
                for k2 in range(H):
                    exh = _lane_gather(ex, bcast[k2])
                    zv = bf[6][ei, pl.ds(k2 * _LANES, _LANES)]
                    bf[7][ei, pl.ds(k2 * _LANES, _LANES)] = zv * exh
                gbase = lax.shift_left(lax.shift_right_logical(ei, 4), 4)
                dv = bf[1][pl.ds(gbase, _LANES)]
                dq = _lane_gather(
                    jnp.bitwise_and(dv, 7).astype(jnp.float32),
                    jnp.full((_LANES,), jnp.bitwise_and(ei, 15), jnp.int32))
                for q in range(8):
                    eqf = 1.0 - jnp.clip(jnp.abs(dq - float(q)), 0.0, 1.0)
                    bf[8][ei, pl.ds(q * _LANES, _LANES)] = ex * eqf
                return ecarry

            lax.fori_loop(0, _B, edge_body, 0)

        def scatters(bf):
            pltpu.sync_copy(bf[8], den_s.at[bf[3]], add=True)
            pltpu.sync_copy(bf[7], acc_s.at[bf[2]], add=True)

        @pl.when(s < NCHUNK)
        def _():
            issue_idx(0, bufs[0])
            wait_idx(bufs[0])
            issue_gathers(bufs[0])

        def pair_body(j, carry):
            for b in (0, 1):
                tt = 2 * j + b
                ch = s + tt * _NS

                @pl.when(ch < NCHUNK)
                def _():
                    bf = bufs[b]
                    nf = bufs[1 - b]

                    @pl.when(ch + _NS < NCHUNK)
                    def _():
                        issue_idx(tt + 1, nf)

                    wait_gathers(bf)
                    compute(bf)
                    scatters(bf)

                    @pl.when(ch + _NS < NCHUNK)
                    def _():
                        wait_idx(nf)
                        issue_gathers(nf)

            return carry

        lax.fori_loop(0, NPAIR, pair_body, 0)
        plsc.subcore_barrier()

        out_base = pl.multiple_of(c * AROWS + s * ZA, 8)
        pltpu.sync_copy(acc_s.at[pl.ds(s * ZA, ZA)],
                        acc_o.at[pl.ds(out_base, ZA)])
        dout_base = pl.multiple_of(c * DROWS + s * ZD, 8)
        pltpu.sync_copy(den_s.at[pl.ds(s * ZD, ZD)],
                        den_o.at[pl.ds(dout_base, ZD)])

    mesh = plsc.VectorSubcoreMesh(core_axis_name="c", subcore_axis_name="s")

    def idxt():
        return pltpu.VMEM((_B,), jnp.int32)

    def rowt(w):
        return pltpu.VMEM((_B, w), jnp.float32)

    acc2, den2 = pl.kernel(
        sc_body,
        out_type=[
            jax.ShapeDtypeStruct((_NC * AROWS, HO), jnp.float32),
            jax.ShapeDtypeStruct((_NC * DROWS, HO), jnp.float32),
        ],
        mesh=mesh,
        scratch_types=[
            idxt(), idxt(), idxt(), idxt(),
            rowt(STW), rowt(STW), rowt(HO), rowt(HO), rowt(HO),
            idxt(), idxt(), idxt(), idxt(),
            rowt(STW), rowt(STW), rowt(HO), rowt(HO), rowt(HO),
            pltpu.VMEM_SHARED((AROWS, HO), jnp.float32),
            pltpu.VMEM_SHARED((DROWS, HO), jnp.float32),
            pltpu.SemaphoreType.DMA,
            pltpu.SemaphoreType.DMA,
            pltpu.SemaphoreType.DMA,
            pltpu.SemaphoreType.DMA,
        ],
    )(src, dst, st, z_flat)

    acc_n = jnp.concatenate([acc2[:HALF], acc2[AROWS:AROWS + HALF]])
    dpk = HALF // 8
    den_n = jnp.concatenate([
        den2[:dpk].reshape(HALF, SW),
        den2[DROWS:DROWS + dpk].reshape(HALF, SW),
    ])

    cc = jnp.arange(HO)
    P = (jnp.arange(SW)[:, None] == (cc // O)[None, :]).astype(jnp.float32)
    h_out = pl.pallas_call(
        _comb_body,
        grid=(N // bn,),
        in_specs=[
            pl.BlockSpec((bn, D), lambda i: (i, 0)),
            pl.BlockSpec((bn, HO), lambda i: (i, 0)),
            pl.BlockSpec((bn, SW), lambda i: (i, 0)),
            pl.BlockSpec((SW, HO), lambda i: (0, 0)),
        ],
        out_specs=pl.BlockSpec((bn, D), lambda i: (i, 0)),
        out_shape=jax.ShapeDtypeStruct((N, D), jnp.float32),
    )(h, acc_n, den_n, P)

    return (h_out, e)


# speedup vs baseline: 1.3905x; 1.3905x over previous
import jax
import jax.numpy as jnp
from jax import lax
from jax.experimental import pallas as pl
from jax.experimental.pallas import tpu as pltpu
from jax.experimental.pallas import tpu_sc as plsc

_B = 128
_NC = 2
_NS = 16
_LANES = 16


def _lane_gather(vec, idx):
    return lax.gather(
        vec,
        idx[:, None],
        dimension_numbers=lax.GatherDimensionNumbers(
            offset_dims=(), collapsed_slice_dims=(0,), start_index_map=(0,)),
        slice_sizes=(1,),
        mode=lax.GatherScatterMode.PROMISE_IN_BOUNDS,
    )


def _proj_body(h_ref, wz_ref, a_ref, z_ref, st_ref):
    hb = h_ref[...]
    z = jnp.dot(hb, wz_ref[...], preferred_element_type=jnp.float32)
    z_ref[...] = z
    st_ref[...] = jnp.dot(z, a_ref[...], preferred_element_type=jnp.float32)


def _comb_body(h_ref, a_ref, d_ref, p_ref, o_ref):
    num = a_ref[...]
    d = d_ref[...]
    r = jnp.where(d > 0, 1.0 / jnp.where(d > 0, d, 1.0), 0.0)
    rexp = jnp.dot(r, p_ref[...], preferred_element_type=jnp.float32)
    hag = num * rexp
    o_ref[...] = h_ref[...] + jnp.where(hag > 0, hag, jnp.exp(hag) - 1.0)


def kernel(h, edge_index, e, W_fc, W_attn):
    N, D = h.shape
    E = edge_index.shape[1]
    H, O, _ = W_fc.shape
    HO = H * O
    SW = 2 * H
    STW = HO
    HALF = N // _NC
    DUMA = ((HALF + 127) // 128) * 128
    AROWS = DUMA + 128
    ZA = AROWS // _NS
    DUMD = DUMA // 8
    DROWS = DUMD + 128
    ZD = DROWS // _NS
    NCHUNK = E // _B
    NK = (NCHUNK + _NS - 1) // _NS

    Wz = W_fc.reshape(HO, D).T
    r_idx = jnp.arange(HO)
    hh = r_idx // O
    oo = r_idx % O
    j16 = jnp.arange(SW)
    w1 = W_attn[hh, oo]
    w2 = W_attn[hh, O + oo]
    A = (jnp.where(j16[None, :] == hh[:, None], w1[:, None], 0.0)
         + jnp.where(j16[None, :] == (H + hh)[:, None], w2[:, None], 0.0))
    A = jnp.pad(A, ((0, 0), (0, STW - SW)))

    bn = 1000
    z_flat, st = pl.pallas_call(
        _proj_body,
        grid=(N // bn,),
        in_specs=[
            pl.BlockSpec((bn, D), lambda i: (i, 0)),
            pl.BlockSpec((D, HO), lambda i: (0, 0)),
            pl.BlockSpec((HO, STW), lambda i: (0, 0)),
        ],
        out_specs=[
            pl.BlockSpec((bn, HO), lambda i: (i, 0)),
            pl.BlockSpec((bn, STW), lambda i: (i, 0)),
        ],
        out_shape=[
            jax.ShapeDtypeStruct((N, HO), jnp.float32),
            jax.ShapeDtypeStruct((N, STW), jnp.float32),
        ],
    )(h, Wz, A)

    src = edge_index[0]
    dst = edge_index[1]

    def sc_body(src_h, dst_h, st_h, z_h, acc_o, den_o,
                src_v, dst_v, dstm_v, dstd8_v, sts_v, std_v, zrows_v, den_v,
                acc_s, den_s, sem):
        c = lax.axis_index("c")
        s = lax.axis_index("s")

        def zrow_body(r, zcarry):
            for k2 in range(HO // _LANES):
                zrows_v[r, pl.ds(k2 * _LANES, _LANES)] = jnp.zeros(
                    (_LANES,), jnp.float32)
            return zcarry

        lax.fori_loop(0, _B, zrow_body, 0)
        for blk in range(ZA // _B):
            pltpu.sync_copy(zrows_v, acc_s.at[pl.ds(s * ZA + blk * _B, _B)])
        rem = ZA % _B
        if rem:
            pltpu.sync_copy(
                zrows_v.at[pl.ds(0, rem)],
                acc_s.at[pl.ds(s * ZA + (ZA // _B) * _B, rem)])
        pltpu.sync_copy(zrows_v.at[pl.ds(0, ZD)],
                        den_s.at[pl.ds(s * ZD, ZD)])
        plsc.subcore_barrier()

        def chunk_body(k, carry):
            ch = s + k * _NS

            @pl.when(ch < NCHUNK)
            def _():
                base = ch * _B
                i1 = pltpu.async_copy(src_h.at[pl.ds(base, _B)], src_v, sem)
                i2 = pltpu.async_copy(dst_h.at[pl.ds(base, _B)], dst_v, sem)
                i1.wait()
                i2.wait()
                g1 = pltpu.async_copy(st_h.at[src_v], sts_v, sem)
                g2 = pltpu.async_copy(st_h.at[dst_v], std_v, sem)
                g3 = pltpu.async_copy(z_h.at[src_v], zrows_v, sem)

                def grp_body(g, gcarry):
                    dv = dst_v[pl.ds(g * _LANES, _LANES)]
                    my = dv - c * HALF
                    val = (jnp.clip(my + 1, 0, 1)
                           * jnp.clip(HALF - my, 0, 1))
                    inv = 1 - val
                    dstm_v[pl.ds(g * _LANES, _LANES)] = val * my + inv * DUMA
                    dstd8_v[pl.ds(g * _LANES, _LANES)] = (
                        val * lax.shift_right_logical(my, 3) + inv * DUMD)
                    return gcarry

                lax.fori_loop(0, _B // _LANES, grp_body, 0, unroll=2)
                g1.wait()
                g2.wait()
                g3.wait()

                def edge_body(ei, ecarry):
                    lane = lax.broadcasted_iota(jnp.int32, (_LANES,), 0)
                    rot = (lane + H) % _LANES
                    lanef = lane.astype(jnp.float32)
                    maskf = jnp.clip(float(H) - lanef, 0.0, 1.0)
                    bcast = [jnp.full((_LANES,), k2, jnp.int32)
                             for k2 in range(H)]
                    a = sts_v[ei, pl.ds(0, _LANES)]
                    b = std_v[ei, pl.ds(0, _LANES)]
                    t = a + _lane_gather(b, rot)
                    lr = jnp.minimum(jnp.maximum(t, 0.01 * t), 50.0)
                    ex = jnp.exp(lr) * maskf
                    for k2 in range(H):
                        exh = _lane_gather(ex, bcast[k2])
                        zv = zrows_v[ei, pl.ds(k2 * _LANES, _LANES)]
                        zrows_v[ei, pl.ds(k2 * _LANES, _LANES)] = zv * exh
                    gbase = lax.shift_left(lax.shift_right_logical(ei, 4), 4)
                    dv = dst_v[pl.ds(gbase, _LANES)]
                    dq = _lane_gather(
                        jnp.bitwise_and(dv, 7).astype(jnp.float32),
                        jnp.full((_LANES,), jnp.bitwise_and(ei, 15), jnp.int32))
                    for q in range(8):
                        eqf = 1.0 - jnp.clip(jnp.abs(dq - float(q)), 0.0, 1.0)
                        den_v[ei, pl.ds(q * _LANES, _LANES)] = ex * eqf
                    return ecarry

                lax.fori_loop(0, _B, edge_body, 0)
                s1 = pltpu.async_copy(den_v, den_s.at[dstd8_v], sem, add=True)
                s2 = pltpu.async_copy(zrows_v, acc_s.at[dstm_v], sem, add=True)
                s1.wait()
                s2.wait()

            return carry

        lax.fori_loop(0, NK, chunk_body, 0)
        plsc.subcore_barrier()

        out_base = pl.multiple_of(c * AROWS + s * ZA, 8)
        pltpu.sync_copy(acc_s.at[pl.ds(s * ZA, ZA)],
                        acc_o.at[pl.ds(out_base, ZA)])
        dout_base = pl.multiple_of(c * DROWS + s * ZD, 8)
        pltpu.sync_copy(den_s.at[pl.ds(s * ZD, ZD)],
                        den_o.at[pl.ds(dout_base, ZD)])

    mesh = plsc.VectorSubcoreMesh(core_axis_name="c", subcore_axis_name="s")
    acc2, den2 = pl.kernel(
        sc_body,
        out_type=[
            jax.ShapeDtypeStruct((_NC * AROWS, HO), jnp.float32),
            jax.ShapeDtypeStruct((_NC * DROWS, HO), jnp.float32),
        ],
        mesh=mesh,
        scratch_types=[
            pltpu.VMEM((_B,), jnp.int32),
            pltpu.VMEM((_B,), jnp.int32),
            pltpu.VMEM((_B,), jnp.int32),
            pltpu.VMEM((_B,), jnp.int32),
            pltpu.VMEM((_B, STW), jnp.float32),
            pltpu.VMEM((_B, STW), jnp.float32),
            pltpu.VMEM((_B, HO), jnp.float32),
            pltpu.VMEM((_B, HO), jnp.float32),
            pltpu.VMEM_SHARED((AROWS, HO), jnp.float32),
            pltpu.VMEM_SHARED((DROWS, HO), jnp.float32),
            pltpu.SemaphoreType.DMA,
        ],
    )(src, dst, st, z_flat)

    acc_n = jnp.concatenate([acc2[:HALF], acc2[AROWS:AROWS + HALF]])
    dpk = HALF // 8
    den_n = jnp.concatenate([
        den2[:dpk].reshape(HALF, SW),
        den2[DROWS:DROWS + dpk].reshape(HALF, SW),
    ])

    cc = jnp.arange(HO)
    P = (jnp.arange(SW)[:, None] == (cc // O)[None, :]).astype(jnp.float32)
    h_out = pl.pallas_call(
        _comb_body,
        grid=(N // bn,),
        in_specs=[
            pl.BlockSpec((bn, D), lambda i: (i, 0)),
            pl.BlockSpec((bn, HO), lambda i: (i, 0)),
            pl.BlockSpec((bn, SW), lambda i: (i, 0)),
            pl.BlockSpec((SW, HO), lambda i: (0, 0)),
        ],
        out_specs=pl.BlockSpec((bn, D), lambda i: (i, 0)),
        out_shape=jax.ShapeDtypeStruct((N, D), jnp.float32),
    )(h, acc_n, den_n, P)

    return (h_out, e)
